# baseline (speedup 1.0000x reference)
import jax
import jax.numpy as jnp
from jax.experimental import pallas as pl
from jax.experimental.pallas import tpu as pltpu

NBUF = 4
KEDGE = 4


def _moe_body(x_ref, Wg_ref, bg_ref, We_hbm, be_ref, out_ref, wbuf, sems):
    n_experts = be_ref.shape[0]
    d_in = x_ref.shape[1]
    dq = d_in // KEDGE
    last = n_experts - 1
    last_slot = last % NBUF

    def wfull(e, slot):
        return pltpu.make_async_copy(We_hbm.at[e], wbuf.at[slot],
                                     sems.at[slot, 0])

    def wquarter(e, slot, k):
        sl = pl.ds(k * dq, dq)
        return pltpu.make_async_copy(We_hbm.at[e, sl], wbuf.at[slot, sl],
                                     sems.at[slot, k])

    for k in range(KEDGE):
        wquarter(0, 0, k).start()
    for i in range(1, NBUF):
        wfull(i, i).start()

    logits = jnp.dot(x_ref[...], Wg_ref[...],
                     preferred_element_type=jnp.float32) + bg_ref[...]
    m = jnp.max(logits, axis=-1, keepdims=True)
    ex = jnp.exp(logits - m)
    gates = ex / jnp.sum(ex, axis=-1, keepdims=True)

    out_ref[...] = jnp.dot(gates, be_ref[...],
                           preferred_element_type=jnp.float32)

    eye = jax.lax.broadcasted_iota(jnp.int32, (1, n_experts), 1)

    def gate_col(e):
        return jnp.sum(gates * (eye == e).astype(jnp.float32),
                       axis=1, keepdims=True)

    xg0 = (x_ref[...] * gate_col(0)).astype(jnp.bfloat16)
    for k in range(KEDGE):
        wquarter(0, 0, k).wait()
        out_ref[...] += jnp.dot(
            xg0[:, k * dq:(k + 1) * dq],
            wbuf[0, pl.ds(k * dq, dq)].astype(jnp.bfloat16),
            preferred_element_type=jnp.float32)
    wfull(NBUF, 0).start()

    def step(e, _):
        slot = jax.lax.rem(e, NBUF)
        wfull(e, slot).wait()
        xg = (x_ref[...] * gate_col(e)).astype(jnp.bfloat16)
        out_ref[...] += jnp.dot(xg, wbuf[slot].astype(jnp.bfloat16),
                                preferred_element_type=jnp.float32)
        t = e + NBUF

        @pl.when(t < last)
        def _():
            wfull(t, slot).start()

        @pl.when(t == last)
        def _():
            for k in range(KEDGE):
                wquarter(last, slot, k).start()

        return 0

    jax.lax.fori_loop(1, last, step, 0)

    xgl = (x_ref[...] * gate_col(last)).astype(jnp.bfloat16)
    for k in range(KEDGE):
        wquarter(last, last_slot, k).wait()
        out_ref[...] += jnp.dot(
            xgl[:, k * dq:(k + 1) * dq],
            wbuf[last_slot, pl.ds(k * dq, dq)].astype(jnp.bfloat16),
            preferred_element_type=jnp.float32)


def kernel(x, Wg, bg, We, be):
    T, D = x.shape
    E, _, H = We.shape
    return pl.pallas_call(
        _moe_body,
        in_specs=[
            pl.BlockSpec(memory_space=pltpu.MemorySpace.VMEM),
            pl.BlockSpec(memory_space=pltpu.MemorySpace.VMEM),
            pl.BlockSpec(memory_space=pltpu.MemorySpace.VMEM),
            pl.BlockSpec(memory_space=pltpu.MemorySpace.HBM),
            pl.BlockSpec(memory_space=pltpu.MemorySpace.VMEM),
        ],
        out_specs=pl.BlockSpec(memory_space=pltpu.MemorySpace.VMEM),
        out_shape=jax.ShapeDtypeStruct((T, H), jnp.float32),
        scratch_shapes=[
            pltpu.VMEM((NBUF, D, H), jnp.float32),
            pltpu.SemaphoreType.DMA((NBUF, KEDGE)),
        ],
    )(x, Wg, bg.reshape(1, E), We, be)
